# Initial kernel scaffold; baseline (speedup 1.0000x reference)
import jax
import jax.numpy as jnp
from jax.experimental import pallas as pl


def kernel(Hx, edge_index, W1, b1, W2, b2, W3, b3):
    raise NotImplementedError("write your pallas kernel here")


# speedup vs baseline: 55.2197x; 55.2197x over previous
import functools

import jax
import jax.numpy as jnp
from jax.experimental import pallas as pl
from jax.experimental.pallas import tpu as pltpu

B = 500
NUMK = 100
EPER = 1600
BB = 16
BPAD = 512


def _col(row_vec, eye):
    return jax.lax.dot_general(eye, row_vec, (((1,), (1,)), ((), ())),
                               preferred_element_type=jnp.float32)


def _row(col_vec, eye):
    return jax.lax.dot_general(col_vec, eye, (((0,), (0,)), ((), ())),
                               preferred_element_type=jnp.float32)


def _gcn_body(ew_ref, p_ref, row_ref, col_ref,
              w1_ref, b1_ref, w2_ref, b2_ref, w3_ref, b3_ref,
              out_ref, ut_scr, v_scr):
    f32 = jnp.float32

    @pl.when(pl.program_id(0) == 0)
    def _():
        iota_j = jax.lax.broadcasted_iota(jnp.int32, (NUMK, EPER), 0)
        ut_scr[...] = (iota_j == col_ref[...]).astype(f32)
        iota_i = jax.lax.broadcasted_iota(jnp.int32, (EPER, NUMK), 1)
        v_scr[...] = (iota_i == row_ref[...]).astype(f32)

    eye = (jax.lax.broadcasted_iota(jnp.int32, (NUMK, NUMK), 0)
           == jax.lax.broadcasted_iota(jnp.int32, (NUMK, NUMK), 1)).astype(f32)

    ut = ut_scr[...]
    v = v_scr[...]
    w1 = w1_ref[...]
    b1 = b1_ref[...]
    w2 = w2_ref[...]
    b2 = b2_ref[...]
    w3 = w3_ref[...]
    b3 = b3_ref[...]

    rows = []
    for b in range(BB):
        ew = ew_ref[pl.ds(b, 1), :]
        s = jnp.dot(ut * ew, v, preferred_element_type=f32)
        spi = s + eye
        deg = jnp.sum(spi, axis=1, keepdims=True)
        dis = jnp.where(deg > 0, jax.lax.rsqrt(deg), 0.0)
        x0 = _col(p_ref[pl.ds(b, 1), :], eye)
        q1 = dis * jnp.dot(spi, dis * x0, preferred_element_type=f32)
        h1 = jnp.maximum(q1 * w1 + b1, 0.0)
        t2 = jnp.dot(h1, w2, preferred_element_type=f32)
        q2 = dis * jnp.dot(spi, dis * t2, preferred_element_type=f32)
        h2 = jnp.maximum(q2 + b2, 0.0)
        t3 = jnp.dot(h2, w3, preferred_element_type=f32)
        q3 = dis * jnp.dot(spi, dis * t3, preferred_element_type=f32)
        rows.append(_row(q3 + b3, eye))
    out_ref[...] = jnp.concatenate(rows, axis=0)


@jax.jit
def kernel(Hx, edge_index, W1, b1, W2, b2, W3, b3):
    f32 = jnp.float32
    p = jnp.zeros((BPAD, NUMK), f32).at[:B].set(Hx[:, :NUMK])
    ew = jnp.zeros((BPAD, EPER), f32).at[:B].set(Hx[:, NUMK:NUMK + EPER])
    row = edge_index[0].reshape(EPER, 1)
    col = edge_index[1].reshape(1, EPER)

    grid = (BPAD // BB,)
    out = pl.pallas_call(
        _gcn_body,
        grid=grid,
        in_specs=[
            pl.BlockSpec((BB, EPER), lambda i: (i, 0)),
            pl.BlockSpec((BB, NUMK), lambda i: (i, 0)),
            pl.BlockSpec((EPER, 1), lambda i: (0, 0)),
            pl.BlockSpec((1, EPER), lambda i: (0, 0)),
            pl.BlockSpec((1, 32), lambda i: (0, 0)),
            pl.BlockSpec((1, 32), lambda i: (0, 0)),
            pl.BlockSpec((32, 32), lambda i: (0, 0)),
            pl.BlockSpec((1, 32), lambda i: (0, 0)),
            pl.BlockSpec((32, 1), lambda i: (0, 0)),
            pl.BlockSpec((1, 1), lambda i: (0, 0)),
        ],
        out_specs=pl.BlockSpec((BB, NUMK), lambda i: (i, 0)),
        out_shape=jax.ShapeDtypeStruct((BPAD, NUMK), f32),
        scratch_shapes=[
            pltpu.VMEM((NUMK, EPER), f32),
            pltpu.VMEM((EPER, NUMK), f32),
        ],
    )(ew, p, row, col,
      W1, b1.reshape(1, 32), W2, b2.reshape(1, 32), W3, b3.reshape(1, 1))
    return out[:B]


# speedup vs baseline: 63.3493x; 1.1472x over previous
import functools

import jax
import jax.numpy as jnp
from jax import lax
from jax.experimental import pallas as pl
from jax.experimental.pallas import tpu as pltpu
from jax.experimental.pallas import tpu_sc as plsc

B = 500
NUMK = 100
EPER = 1600
SDIM = NUMK * NUMK
BPAD = 512
BB = 16
NTILES = 32
SPT = BPAD // NTILES


def _sc_build(ew_hbm, ei_hbm, s_hbm, row_v, col_v, idx_v, ew_v, s_v):
    wid = lax.axis_index("s") * 2 + lax.axis_index("c")
    pltpu.sync_copy(ei_hbm.at[0], row_v)
    pltpu.sync_copy(ei_hbm.at[1], col_v)

    def mkidx(k, c):
        sl = pl.ds(k * 16, 16)
        idx_v[sl] = col_v[sl] * NUMK + row_v[sl]
        return c
    lax.fori_loop(0, EPER // 16, mkidx, 0)

    zero16 = jnp.zeros((16,), jnp.float32)

    def per_sample(i, c):
        b = wid * SPT + i
        pltpu.sync_copy(ew_hbm.at[b], ew_v)

        def zloop(k, cc):
            for u in range(25):
                s_v[pl.ds((k * 25 + u) * 16, 16)] = zero16
            return cc
        lax.fori_loop(0, SDIM // (16 * 25), zloop, 0)

        def scat(k, cc):
            for u in range(10):
                sl = pl.ds((k * 10 + u) * 16, 16)
                plsc.addupdate_scatter(s_v, [idx_v[sl]], ew_v[sl])
            return cc
        lax.fori_loop(0, EPER // (16 * 10), scat, 0)

        pltpu.sync_copy(s_v, s_hbm.at[b])
        return c
    lax.fori_loop(0, SPT, per_sample, 0)


def _build_s(ew, edge_index):
    mesh = plsc.VectorSubcoreMesh(core_axis_name="c", subcore_axis_name="s")
    f = pl.kernel(
        _sc_build,
        mesh=mesh,
        compiler_params=pltpu.CompilerParams(needs_layout_passes=False),
        out_type=jax.ShapeDtypeStruct((BPAD, SDIM), jnp.float32),
        scratch_types=[
            pltpu.VMEM((EPER,), jnp.int32),
            pltpu.VMEM((EPER,), jnp.int32),
            pltpu.VMEM((EPER,), jnp.int32),
            pltpu.VMEM((EPER,), jnp.float32),
            pltpu.VMEM((SDIM,), jnp.float32),
        ],
    )
    return f(ew, edge_index)


def _col(row_vec, eye):
    return jax.lax.dot_general(eye, row_vec, (((1,), (1,)), ((), ())),
                               preferred_element_type=jnp.float32)


def _row(col_vec, eye):
    return jax.lax.dot_general(col_vec, eye, (((0,), (0,)), ((), ())),
                               preferred_element_type=jnp.float32)


def _gcn_body(s_ref, p_ref, w1_ref, b1_ref, w2_ref, b2_ref, w3_ref, b3_ref,
              out_ref):
    f32 = jnp.float32
    eye = (jax.lax.broadcasted_iota(jnp.int32, (NUMK, NUMK), 0)
           == jax.lax.broadcasted_iota(jnp.int32, (NUMK, NUMK), 1)).astype(f32)

    w1 = w1_ref[...]
    b1 = b1_ref[...]
    w2 = w2_ref[...]
    b2 = b2_ref[...]
    w3 = w3_ref[...]
    b3 = b3_ref[...]

    rows = []
    for b in range(BB):
        spi = s_ref[b] + eye
        deg = jnp.sum(spi, axis=1, keepdims=True)
        dis = jnp.where(deg > 0, jax.lax.rsqrt(deg), 0.0)
        x0 = _col(p_ref[pl.ds(b, 1), :], eye)
        q1 = dis * jnp.dot(spi, dis * x0, preferred_element_type=f32)
        h1 = jnp.maximum(q1 * w1 + b1, 0.0)
        t2 = jnp.dot(h1, w2, preferred_element_type=f32)
        q2 = dis * jnp.dot(spi, dis * t2, preferred_element_type=f32)
        h2 = jnp.maximum(q2 + b2, 0.0)
        t3 = jnp.dot(h2, w3, preferred_element_type=f32)
        q3 = dis * jnp.dot(spi, dis * t3, preferred_element_type=f32)
        rows.append(_row(q3 + b3, eye))
    out_ref[...] = jnp.concatenate(rows, axis=0)


@jax.jit
def kernel(Hx, edge_index, W1, b1, W2, b2, W3, b3):
    f32 = jnp.float32
    p = jnp.zeros((BPAD, NUMK), f32).at[:B].set(Hx[:, :NUMK])
    ew = jnp.zeros((BPAD, EPER), f32).at[:B].set(Hx[:, NUMK:NUMK + EPER])

    s3 = _build_s(ew, edge_index).reshape(BPAD, NUMK, NUMK)

    grid = (BPAD // BB,)
    out = pl.pallas_call(
        _gcn_body,
        grid=grid,
        in_specs=[
            pl.BlockSpec((BB, NUMK, NUMK), lambda i: (i, 0, 0)),
            pl.BlockSpec((BB, NUMK), lambda i: (i, 0)),
            pl.BlockSpec((1, 32), lambda i: (0, 0)),
            pl.BlockSpec((1, 32), lambda i: (0, 0)),
            pl.BlockSpec((32, 32), lambda i: (0, 0)),
            pl.BlockSpec((1, 32), lambda i: (0, 0)),
            pl.BlockSpec((32, 1), lambda i: (0, 0)),
            pl.BlockSpec((1, 1), lambda i: (0, 0)),
        ],
        out_specs=pl.BlockSpec((BB, NUMK), lambda i: (i, 0)),
        out_shape=jax.ShapeDtypeStruct((BPAD, NUMK), f32),
    )(s3, p,
      W1, b1.reshape(1, 32), W2, b2.reshape(1, 32), W3, b3.reshape(1, 1))
    return out[:B]


# speedup vs baseline: 200.6872x; 3.1679x over previous
import functools

import jax
import jax.numpy as jnp
from jax import lax
from jax.experimental import pallas as pl
from jax.experimental.pallas import tpu as pltpu
from jax.experimental.pallas import tpu_sc as plsc

B = 500
NUMK = 100
EPER = 1600
SDIM = NUMK * NUMK
BPAD = 512
BB = 16
NTILES = 32
SPT = BPAD // NTILES


def _sc_build(ew_hbm, ei_hbm, s_hbm, row_v, col_v, idx_v, ew_v, s_v):
    wid = lax.axis_index("s") * 2 + lax.axis_index("c")
    pltpu.sync_copy(ei_hbm.at[0], row_v)
    pltpu.sync_copy(ei_hbm.at[1], col_v)

    def mkidx(k, c):
        sl = pl.ds(k * 16, 16)
        idx_v[sl] = col_v[sl] * NUMK + row_v[sl]
        return c
    lax.fori_loop(0, EPER // 16, mkidx, 0)

    zero16 = jnp.zeros((16,), jnp.float32)

    def per_sample(i, c):
        b = wid * SPT + i
        pltpu.sync_copy(ew_hbm.at[b], ew_v)

        def zloop(k, cc):
            for u in range(25):
                s_v[pl.ds((k * 25 + u) * 16, 16)] = zero16
            return cc
        lax.fori_loop(0, SDIM // (16 * 25), zloop, 0)

        def scat(k, cc):
            for u in range(10):
                sl = pl.ds((k * 10 + u) * 16, 16)
                plsc.addupdate_scatter(s_v, [idx_v[sl]], ew_v[sl])
            return cc
        lax.fori_loop(0, EPER // (16 * 10), scat, 0)

        pltpu.sync_copy(s_v, s_hbm.at[b])
        return c
    lax.fori_loop(0, SPT, per_sample, 0)


def _build_s(ew, edge_index):
    mesh = plsc.VectorSubcoreMesh(core_axis_name="c", subcore_axis_name="s")
    f = pl.kernel(
        _sc_build,
        mesh=mesh,
        compiler_params=pltpu.CompilerParams(needs_layout_passes=False),
        out_type=jax.ShapeDtypeStruct((BPAD, SDIM), jnp.float32),
        scratch_types=[
            pltpu.VMEM((EPER,), jnp.int32),
            pltpu.VMEM((EPER,), jnp.int32),
            pltpu.VMEM((EPER,), jnp.int32),
            pltpu.VMEM((EPER,), jnp.float32),
            pltpu.VMEM((SDIM,), jnp.float32),
        ],
    )
    return f(ew, edge_index)


def _col(row_vec, eye):
    return jax.lax.dot_general(eye, row_vec, (((1,), (1,)), ((), ())),
                               preferred_element_type=jnp.float32)


def _row(col_vec, eye):
    return jax.lax.dot_general(col_vec, eye, (((0,), (0,)), ((), ())),
                               preferred_element_type=jnp.float32)


def _gcn_body(s_ref, p_ref, w1_ref, b1_ref, w2_ref, b2_ref, w3_ref, b3_ref,
              out_ref):
    f32 = jnp.float32
    eye = (jax.lax.broadcasted_iota(jnp.int32, (NUMK, NUMK), 0)
           == jax.lax.broadcasted_iota(jnp.int32, (NUMK, NUMK), 1)).astype(f32)

    w1 = w1_ref[...]
    b1 = b1_ref[...]
    w2 = w2_ref[...]
    b2 = b2_ref[...]
    w3 = w3_ref[...]
    b3 = b3_ref[...]

    R = range(BB)
    spi = [s_ref[b] + eye for b in R]
    deg = [jnp.sum(spi[b], axis=1, keepdims=True) for b in R]
    dis = [jnp.where(deg[b] > 0, jax.lax.rsqrt(deg[b]), 0.0) for b in R]
    x0 = [_col(p_ref[pl.ds(b, 1), :], eye) for b in R]
    q1 = [dis[b] * jnp.dot(spi[b], dis[b] * x0[b], preferred_element_type=f32)
          for b in R]
    h1 = [jnp.maximum(q1[b] * w1 + b1, 0.0) for b in R]
    t2 = [jnp.dot(h1[b], w2, preferred_element_type=f32) for b in R]
    q2 = [dis[b] * jnp.dot(spi[b], dis[b] * t2[b], preferred_element_type=f32)
          for b in R]
    h2 = [jnp.maximum(q2[b] + b2, 0.0) for b in R]
    t3 = [jnp.dot(h2[b], w3, preferred_element_type=f32) for b in R]
    q3 = [dis[b] * jnp.dot(spi[b], dis[b] * t3[b], preferred_element_type=f32)
          for b in R]
    rows = [_row(q3[b] + b3, eye) for b in R]
    out_ref[...] = jnp.concatenate(rows, axis=0)


@jax.jit
def kernel(Hx, edge_index, W1, b1, W2, b2, W3, b3):
    f32 = jnp.float32
    p = jnp.zeros((BPAD, NUMK), f32).at[:B].set(Hx[:, :NUMK])
    ew = jnp.zeros((BPAD, EPER), f32).at[:B].set(Hx[:, NUMK:NUMK + EPER])

    s3 = _build_s(ew, edge_index).reshape(BPAD, NUMK, NUMK)

    grid = (BPAD // BB,)
    out = pl.pallas_call(
        _gcn_body,
        grid=grid,
        in_specs=[
            pl.BlockSpec((BB, NUMK, NUMK), lambda i: (i, 0, 0)),
            pl.BlockSpec((BB, NUMK), lambda i: (i, 0)),
            pl.BlockSpec((1, 32), lambda i: (0, 0)),
            pl.BlockSpec((1, 32), lambda i: (0, 0)),
            pl.BlockSpec((32, 32), lambda i: (0, 0)),
            pl.BlockSpec((1, 32), lambda i: (0, 0)),
            pl.BlockSpec((32, 1), lambda i: (0, 0)),
            pl.BlockSpec((1, 1), lambda i: (0, 0)),
        ],
        out_specs=pl.BlockSpec((BB, NUMK), lambda i: (i, 0)),
        out_shape=jax.ShapeDtypeStruct((BPAD, NUMK), f32),
    )(s3, p,
      W1, b1.reshape(1, 32), W2, b2.reshape(1, 32), W3, b3.reshape(1, 1))
    return out[:B]


# speedup vs baseline: 230.2344x; 1.1472x over previous
import functools

import jax
import jax.numpy as jnp
from jax import lax
from jax.experimental import pallas as pl
from jax.experimental.pallas import tpu as pltpu
from jax.experimental.pallas import tpu_sc as plsc

B = 500
NUMK = 100
EPER = 1600
BPAD = 512
BB = 16
NTILES = 32
SPT = BPAD // NTILES


def _sc_build(ew_hbm, ei_hbm, z_hbm, s_hbm, row_v, col_v, ew_v, s_v):
    wid = lax.axis_index("s") * 2 + lax.axis_index("c")
    pltpu.sync_copy(ei_hbm.at[0], row_v)
    pltpu.sync_copy(ei_hbm.at[1], col_v)
    pltpu.sync_copy(z_hbm, s_v)
    base = wid * SPT

    def per_sample(i, c):
        b = base + i

        @pl.when(b < B)
        def _():
            pltpu.sync_copy(ew_hbm.at[b], ew_v)

            def scat(k, cc):
                for u in range(10):
                    sl = pl.ds((k * 10 + u) * 16, 16)
                    plsc.addupdate_scatter(
                        s_v, [col_v[sl], row_v[sl]], ew_v[sl])
                return cc
            lax.fori_loop(0, EPER // 160, scat, 0)

            pltpu.sync_copy(s_v, s_hbm.at[b])

            def unscat(k, cc):
                for u in range(10):
                    sl = pl.ds((k * 10 + u) * 16, 16)
                    plsc.addupdate_scatter(
                        s_v, [col_v[sl], row_v[sl]], -ew_v[sl])
                return cc
            lax.fori_loop(0, EPER // 160, unscat, 0)
        return c
    lax.fori_loop(0, SPT, per_sample, 0)


def _build_s(ew, edge_index):
    mesh = plsc.VectorSubcoreMesh(core_axis_name="c", subcore_axis_name="s")
    f = pl.kernel(
        _sc_build,
        mesh=mesh,
        compiler_params=pltpu.CompilerParams(needs_layout_passes=False),
        out_type=jax.ShapeDtypeStruct((BPAD, NUMK, NUMK), jnp.float32),
        scratch_types=[
            pltpu.VMEM((EPER,), jnp.int32),
            pltpu.VMEM((EPER,), jnp.int32),
            pltpu.VMEM((EPER,), jnp.float32),
            pltpu.VMEM((NUMK, NUMK), jnp.float32),
        ],
    )
    zeros = jnp.zeros((NUMK, NUMK), jnp.float32)
    return f(ew, edge_index, zeros)


def _gcn_body(s_ref, p_ref, w1_ref, b1_ref, w2_ref, b2_ref, w3_ref, b3_ref,
              out_ref):
    f32 = jnp.float32
    eye = (jax.lax.broadcasted_iota(jnp.int32, (NUMK, NUMK), 0)
           == jax.lax.broadcasted_iota(jnp.int32, (NUMK, NUMK), 1)).astype(f32)

    w1 = w1_ref[...]
    b1 = b1_ref[...]
    w2 = w2_ref[...]
    b2 = b2_ref[...]
    w3 = w3_ref[...]
    b3 = b3_ref[...]

    R = range(BB)
    spi = [s_ref[b] + eye for b in R]
    deg = [jnp.sum(spi[b], axis=1, keepdims=True) for b in R]
    dis = [jnp.where(deg[b] > 0, jax.lax.rsqrt(deg[b]), 0.0) for b in R]
    x0 = [jnp.transpose(p_ref[pl.ds(b, 1), :]) for b in R]
    q1 = [dis[b] * jnp.dot(spi[b], dis[b] * x0[b], preferred_element_type=f32)
          for b in R]
    h1 = [jnp.maximum(q1[b] * w1 + b1, 0.0) for b in R]
    t2 = [jnp.dot(h1[b], w2, preferred_element_type=f32) for b in R]
    q2 = [dis[b] * jnp.dot(spi[b], dis[b] * t2[b], preferred_element_type=f32)
          for b in R]
    h2 = [jnp.maximum(q2[b] + b2, 0.0) for b in R]
    t3 = [jnp.dot(h2[b], w3, preferred_element_type=f32) for b in R]
    q3 = [dis[b] * jnp.dot(spi[b], dis[b] * t3[b], preferred_element_type=f32)
          for b in R]
    rows = [jnp.transpose(q3[b] + b3) for b in R]
    out_ref[...] = jnp.concatenate(rows, axis=0)


@jax.jit
def kernel(Hx, edge_index, W1, b1, W2, b2, W3, b3):
    f32 = jnp.float32
    p = jnp.zeros((BPAD, NUMK), f32).at[:B].set(Hx[:, :NUMK])
    ew = Hx[:, NUMK:NUMK + EPER]

    s3 = _build_s(ew, edge_index)

    grid = (BPAD // BB,)
    out = pl.pallas_call(
        _gcn_body,
        grid=grid,
        in_specs=[
            pl.BlockSpec((BB, NUMK, NUMK), lambda i: (i, 0, 0)),
            pl.BlockSpec((BB, NUMK), lambda i: (i, 0)),
            pl.BlockSpec((1, 32), lambda i: (0, 0)),
            pl.BlockSpec((1, 32), lambda i: (0, 0)),
            pl.BlockSpec((32, 32), lambda i: (0, 0)),
            pl.BlockSpec((1, 32), lambda i: (0, 0)),
            pl.BlockSpec((32, 1), lambda i: (0, 0)),
            pl.BlockSpec((1, 1), lambda i: (0, 0)),
        ],
        out_specs=pl.BlockSpec((BB, NUMK), lambda i: (i, 0)),
        out_shape=jax.ShapeDtypeStruct((BPAD, NUMK), f32),
    )(s3, p,
      W1, b1.reshape(1, 32), W2, b2.reshape(1, 32), W3, b3.reshape(1, 1))
    return out[:B]
